# baseline (speedup 1.0000x reference)
import jax
import jax.numpy as jnp
from jax import lax
from jax.experimental import pallas as pl
from jax.experimental.pallas import tpu as pltpu
from jax.experimental.pallas import tpu_sc as plsc

E_TOTAL = 100000
R_TOTAL = 500
DIM = 128
C = 5
B = 128
NES = 4 * B
NRS = 2 * B
NF = NES * 36
NSEG = 1160
WCH = 32
NWBUF = 4
NTR = 36
NW = 32
EPW = NES // NW
CHUNK = 96
NCH = NF // (NW * CHUNK)


def _wid():
    return lax.axis_index("s") * 2 + lax.axis_index("c")


def _sc_build_h(all_e_hbm, adjc_hbm, emb_hbm, ect_hbm,
                all_r_hbm, radjc_hbm, remb_hbm, rct_hbm,
                h_ent_hbm, h_rel_a_hbm, h_rel_b_hbm,
                idx_v, rows_v, rows2_v, sem):
    wid = _wid()
    iota = lax.iota(jnp.int32, 16)
    base = wid * EPW
    pltpu.sync_copy(all_e_hbm.at[pl.ds(base, 16)], idx_v)
    tgt0 = (iota + base) * 6
    pltpu.async_copy(emb_hbm.at[idx_v], rows_v, sem).wait()
    pltpu.async_copy(rows_v, h_ent_hbm.at[tgt0], sem).wait()
    for c in range(C):
        pltpu.sync_copy(adjc_hbm.at[pl.ds(c * NES + base, 16)], idx_v)
        pltpu.async_copy(ect_hbm.at[idx_v], rows_v, sem).wait()
        pltpu.async_copy(rows_v, h_ent_hbm.at[tgt0 + 1 + c], sem).wait()

    @pl.when(wid < 16)
    def _():
        rbase = wid * 16
        pltpu.sync_copy(all_r_hbm.at[pl.ds(rbase, 16)], idx_v)
        rtgt = rbase + iota
        pltpu.async_copy(remb_hbm.at[idx_v], rows_v, sem).wait()
        pltpu.async_copy(rows_v, h_rel_a_hbm.at[rtgt], sem).wait()
        zcol = jnp.full((16,), R_TOTAL, jnp.int32)
        pltpu.async_copy(rct_hbm.at[zcol], rows2_v, sem).wait()
        pltpu.async_copy(rows2_v, h_rel_b_hbm.at[rtgt], sem).wait()
        for c in range(C):
            pltpu.sync_copy(radjc_hbm.at[pl.ds(2 * c * NRS + rbase, 16)], idx_v)
            pltpu.async_copy(rct_hbm.at[idx_v], rows_v, sem).wait()
            pltpu.async_copy(rows_v, h_rel_a_hbm.at[(1 + c) * NRS + rtgt], sem).wait()
            pltpu.sync_copy(radjc_hbm.at[pl.ds((2 * c + 1) * NRS + rbase, 16)], idx_v)
            pltpu.async_copy(rct_hbm.at[idx_v], rows2_v, sem).wait()
            pltpu.async_copy(rows2_v, h_rel_b_hbm.at[(1 + c) * NRS + rtgt], sem).wait()


def _sc_gather_rows(src_idx_hbm, h_ent_hbm, xs_hbm, idx_v, rows_v, sem):
    wid = _wid()
    pltpu.sync_copy(src_idx_hbm.at[wid], idx_v)
    for j in range(NCH):
        p = (wid * NCH + j) * CHUNK
        pltpu.async_copy(h_ent_hbm.at[idx_v.at[j]], rows_v, sem).wait()
        pltpu.sync_copy(rows_v, xs_hbm.at[pl.ds(p, CHUNK)])


def _sc_scatter_rows(tgt_idx_hbm, ys_hbm, ynat_hbm, idx_v, rows_v, sem):
    wid = _wid()
    pltpu.sync_copy(tgt_idx_hbm.at[wid], idx_v)
    for j in range(NCH):
        p = (wid * NCH + j) * CHUNK
        pltpu.sync_copy(ys_hbm.at[pl.ds(p, CHUNK)], rows_v)
        pltpu.async_copy(rows_v, ynat_hbm.at[idx_v.at[j]], sem).wait()


def _tc_stream_mm(fs_ref, loc_ref, end_ref, wl_ref, sl_ref, ftr_ref, ow_ref,
                  os_ref, fst_ref, ssl_ref, poff_ref, pok_ref,
                  xs_ref, d_ref, w_hbm, ys_ref, wbufs, sems):
    t = pl.program_id(0)

    @pl.when(t == 0)
    def _():
        for j in range(NWBUF):
            @pl.when(pok_ref[j] == 1)
            def _(j=j):
                pltpu.make_async_copy(w_hbm.at[pl.ds(poff_ref[j], WCH)],
                                      wbufs.at[j], sems.at[j]).start()

    s0 = fs_ref[t]
    n = fs_ref[t + 1] - s0
    x = xs_ref[pl.ds(t * 128, 128), :] * d_ref[pl.ds(t * 128, 128), :]
    rid = lax.broadcasted_iota(jnp.int32, (128, 1), 0)

    def seg(q, acc):
        @pl.when(ftr_ref[q] == 1)
        def _():
            pltpu.make_async_copy(w_hbm.at[pl.ds(ow_ref[q], WCH)],
                                  wbufs.at[sl_ref[q]],
                                  sems.at[sl_ref[q]]).wait()

            @pl.when(fst_ref[q] == 1)
            def _():
                pltpu.make_async_copy(w_hbm.at[pl.ds(os_ref[q], WCH)],
                                      wbufs.at[ssl_ref[q]],
                                      sems.at[ssl_ref[q]]).start()

        a = loc_ref[q]
        b = end_ref[q]
        xm = jnp.where((rid >= a) & (rid < b), x, 0.0)
        return acc + jnp.dot(xm, wbufs[sl_ref[q], wl_ref[q]],
                             preferred_element_type=jnp.float32)

    def body2(it, accs):
        a0, a1 = accs
        return (seg(s0 + 2 * it, a0), seg(s0 + 2 * it + 1, a1))

    z = jnp.zeros((128, DIM), jnp.float32)
    half = n // 2
    a0, a1 = lax.fori_loop(0, half, body2, (z, z))
    acc = a0 + a1
    acc = lax.cond(n % 2 == 1, lambda: seg(s0 + n - 1, acc), lambda: acc)
    ys_ref[pl.ds(t * 128, 128), :] = acc


def _tc_reduce_relu(yn_ref, out_ref):
    acc = yn_ref[0]
    for k in range(1, 6):
        acc = acc + yn_ref[k]
    out_ref[...] = jnp.maximum(acc, 0.0)


def _tc_rel_gcn(at_ref, ha_ref, hb_ref, wr_ref, out_ref):
    hk = [ha_ref[k] + hb_ref[k] for k in range(6)]
    for j in range(6):
        sup = jnp.zeros((NRS, DIM), jnp.float32)
        for k in range(6):
            ajk = at_ref[j, k, :]
            sup = sup + ajk[:, None] * hk[k]
        out_ref[j] = jnp.maximum(
            jnp.dot(sup, wr_ref[...], preferred_element_type=jnp.float32), 0.0)


def kernel(epoch, pos_h, pos_r, pos_t, neg_h, neg_r, neg_t, ph_R, ph_D, ph_nn,
           pr_A, pt_R, pt_D, pt_nn, nh_R, nh_D, nh_nn, nr_A, nt_R, nt_D, nt_nn,
           entity_emb, relation_emb, entity_context_table,
           relation_context_table, entity_gcn_weight, relation_gcn_weight,
           entity_adj_table, relation_adj_table):
    f32 = jnp.float32
    i32 = jnp.int32

    all_e = jnp.concatenate([pos_h, pos_t, neg_h, neg_t]).astype(i32)
    all_r = jnp.concatenate([pos_r, neg_r]).astype(i32)
    adjc = entity_adj_table[all_e].astype(i32).T.reshape(-1)
    radjc = relation_adj_table[all_r].astype(i32).T.reshape(-1)

    rel_flat = jnp.clip(
        jnp.concatenate([ph_R, pt_R, nh_R, nt_R]).reshape(-1).astype(i32),
        0, 2 * R_TOTAL)
    d_flat = jnp.concatenate([ph_D, pt_D, nh_D, nt_D]).reshape(-1)
    order = jnp.argsort(rel_flat).astype(i32)
    sorted_rel = rel_flat[order]
    d_sorted = d_flat[order].reshape(NF, 1)
    src_sorted = ((order // 36) * 6 + order % 6).astype(i32).reshape(
        NW, NCH, CHUNK)
    tgt_sorted = ((order % 6) * (NES * 6) + (order // 36) * 6
                  + (order % 36) // 6).astype(i32).reshape(NW, NCH, CHUNK)

    ii = jnp.arange(NF, dtype=i32)
    change = jnp.concatenate(
        [jnp.ones((1,), bool), sorted_rel[1:] != sorted_rel[:-1]])
    flag = change | (ii % 128 == 0)
    starts = jnp.nonzero(flag, size=NSEG, fill_value=NF)[0].astype(i32)
    seg_rel = jnp.where(starts < NF,
                        sorted_rel[jnp.clip(starts, 0, NF - 1)],
                        2 * R_TOTAL).astype(i32)
    seg_loc = (starts % 128).astype(i32)
    ends = jnp.concatenate([starts[1:], jnp.array([NF], i32)])
    seg_end = seg_loc + (ends - starts)
    first_seg = jnp.searchsorted(
        starts, jnp.arange(NF // 128 + 1, dtype=i32) * 128).astype(i32)
    cs = seg_rel // WCH
    off = jnp.minimum(cs * WCH, 2 * R_TOTAL + 1 - WCH).astype(i32)
    swloc = (seg_rel - off).astype(i32)
    ftrans = jnp.concatenate(
        [jnp.ones((1,), i32), (cs[1:] != cs[:-1]).astype(i32)])
    k_of = jnp.cumsum(ftrans).astype(i32) - 1
    kmax = k_of[-1]
    sslot = (k_of % NWBUF).astype(i32)
    tr_idx = jnp.nonzero(ftrans, size=NTR, fill_value=NSEG - 1)[0]
    seq_off = off[tr_idx]
    sostart = seq_off[jnp.clip(k_of + NWBUF - 1, 0, NTR - 1)].astype(i32)
    sfstart = (ftrans.astype(bool) & (k_of + NWBUF - 1 <= kmax)
               & (k_of >= 1)).astype(i32)
    sstart_slot = ((k_of + NWBUF - 1) % NWBUF).astype(i32)
    prime_off = seq_off[:NWBUF].astype(i32)
    prime_ok = (jnp.arange(NWBUF) <= kmax).astype(i32)

    mesh = plsc.VectorSubcoreMesh(core_axis_name="c", subcore_axis_name="s")

    h_ent, h_rel_a, h_rel_b = pl.kernel(
        _sc_build_h,
        out_type=[jax.ShapeDtypeStruct((NES * 6, DIM), f32),
                  jax.ShapeDtypeStruct((6 * NRS, DIM), f32),
                  jax.ShapeDtypeStruct((6 * NRS, DIM), f32)],
        mesh=mesh,
        scratch_types=[pltpu.VMEM((16,), i32),
                       pltpu.VMEM((16, DIM), f32),
                       pltpu.VMEM((16, DIM), f32),
                       pltpu.SemaphoreType.DMA],
    )(all_e, adjc, entity_emb, entity_context_table,
      all_r, radjc, relation_emb, relation_context_table)

    xs = pl.kernel(
        _sc_gather_rows,
        out_type=jax.ShapeDtypeStruct((NF, DIM), f32),
        mesh=mesh,
        scratch_types=[pltpu.VMEM((NCH, CHUNK), i32),
                       pltpu.VMEM((CHUNK, DIM), f32),
                       pltpu.SemaphoreType.DMA],
    )(src_sorted, h_ent)

    grid_spec = pltpu.PrefetchScalarGridSpec(
        num_scalar_prefetch=12,
        grid=(NF // 128,),
        in_specs=[
            pl.BlockSpec((NF, DIM), lambda i, *_: (0, 0)),
            pl.BlockSpec((NF, 1), lambda i, *_: (0, 0)),
            pl.BlockSpec(memory_space=pl.ANY),
        ],
        out_specs=pl.BlockSpec((NF, DIM), lambda i, *_: (0, 0)),
        scratch_shapes=[pltpu.VMEM((NWBUF, WCH, DIM, DIM), f32),
                        pltpu.SemaphoreType.DMA((NWBUF,))],
    )
    ys = pl.pallas_call(
        _tc_stream_mm,
        grid_spec=grid_spec,
        out_shape=jax.ShapeDtypeStruct((NF, DIM), f32),
        compiler_params=pltpu.CompilerParams(
            dimension_semantics=("arbitrary",)),
    )(first_seg, seg_loc, seg_end, swloc, sslot, ftrans, off,
      sostart, sfstart, sstart_slot, prime_off, prime_ok,
      xs, d_sorted, entity_gcn_weight)

    ynat = pl.kernel(
        _sc_scatter_rows,
        out_type=jax.ShapeDtypeStruct((NF, DIM), f32),
        mesh=mesh,
        scratch_types=[pltpu.VMEM((NCH, CHUNK), i32),
                       pltpu.VMEM((CHUNK, DIM), f32),
                       pltpu.SemaphoreType.DMA],
    )(tgt_sorted, ys)

    yn3 = ynat.reshape(6, NES * 6, DIM)
    ent_out = pl.pallas_call(
        _tc_reduce_relu,
        grid=(8,),
        in_specs=[pl.BlockSpec((6, NES * 6 // 8, DIM), lambda i: (0, i, 0))],
        out_specs=pl.BlockSpec((NES * 6 // 8, DIM), lambda i: (i, 0)),
        out_shape=jax.ShapeDtypeStruct((NES * 6, DIM), f32),
    )(yn3)
    eo = ent_out.reshape(4, B, 6, DIM)

    a_t = jnp.concatenate([pr_A, nr_A]).transpose(1, 2, 0)
    rel_out = pl.pallas_call(
        _tc_rel_gcn,
        out_shape=jax.ShapeDtypeStruct((6, NRS, DIM), f32),
    )(a_t, h_rel_a.reshape(6, NRS, DIM), h_rel_b.reshape(6, NRS, DIM),
      relation_gcn_weight)
    ro = rel_out.transpose(1, 0, 2)

    return (eo[0], eo[1], eo[2], eo[3], ro[:B], ro[B:])


# speedup vs baseline: 1.0663x; 1.0663x over previous
import jax
import jax.numpy as jnp
from jax import lax
from jax.experimental import pallas as pl
from jax.experimental.pallas import tpu as pltpu
from jax.experimental.pallas import tpu_sc as plsc

E_TOTAL = 100000
R_TOTAL = 500
DIM = 128
C = 5
B = 128
NES = 4 * B
NRS = 2 * B
NF = NES * 36
NSEG = 1160
WCH = 32
NWBUF = 4
NTR = 36
NW = 32
EPW = NES // NW
CHUNK = 96
NCH = NF // (NW * CHUNK)


def _wid():
    return lax.axis_index("s") * 2 + lax.axis_index("c")


def _sc_build_h(all_e_hbm, adjc2_hbm, emb_hbm, ect_hbm,
                all_r_hbm, radjc2_hbm, remb_hbm, rct_hbm,
                h_ent_hbm, h_rel_a_hbm, h_rel_b_hbm,
                idx_v, idx5_v, idx10_v, buf, sem):
    wid = _wid()
    iota = lax.iota(jnp.int32, 16)
    base = wid * EPW
    pltpu.sync_copy(all_e_hbm.at[pl.ds(base, 16)], idx_v)
    pltpu.sync_copy(adjc2_hbm.at[wid], idx5_v)
    tgt0 = (iota + base) * 6

    def g_ent(c):
        if c == 0:
            return pltpu.make_async_copy(emb_hbm.at[idx_v], buf.at[0], sem)
        return pltpu.make_async_copy(ect_hbm.at[idx5_v.at[c - 1]], buf.at[c],
                                     sem)

    def s_ent(c):
        return pltpu.make_async_copy(buf.at[c], h_ent_hbm.at[tgt0 + c], sem)

    for c in range(6):
        g_ent(c).start()
    for c in range(6):
        g_ent(c).wait()
    for c in range(6):
        s_ent(c).start()
    for c in range(6):
        s_ent(c).wait()

    @pl.when(wid < 16)
    def _():
        rbase = wid * 16
        pltpu.sync_copy(all_r_hbm.at[pl.ds(rbase, 16)], idx_v)
        pltpu.sync_copy(radjc2_hbm.at[wid], idx10_v)
        rtgt = rbase + iota
        zcol = jnp.full((16,), R_TOTAL, jnp.int32)

        def g_rel(j):
            if j == 0:
                return pltpu.make_async_copy(remb_hbm.at[idx_v], buf.at[0],
                                             sem)
            if j == 11:
                return pltpu.make_async_copy(rct_hbm.at[zcol], buf.at[11],
                                             sem)
            return pltpu.make_async_copy(rct_hbm.at[idx10_v.at[j - 1]],
                                         buf.at[j], sem)

        def s_rel(j):
            if j == 0:
                dst = h_rel_a_hbm.at[rtgt]
            elif j == 11:
                dst = h_rel_b_hbm.at[rtgt]
            elif j % 2 == 1:
                dst = h_rel_a_hbm.at[(1 + (j - 1) // 2) * NRS + rtgt]
            else:
                dst = h_rel_b_hbm.at[(1 + (j - 2) // 2) * NRS + rtgt]
            return pltpu.make_async_copy(buf.at[j], dst, sem)

        for j in range(12):
            g_rel(j).start()
        for j in range(12):
            g_rel(j).wait()
        for j in range(12):
            s_rel(j).start()
        for j in range(12):
            s_rel(j).wait()


def _sc_gather_rows(src_idx_hbm, h_ent_hbm, xs_hbm, idx_v, rows_v, sem):
    wid = _wid()
    pltpu.sync_copy(src_idx_hbm.at[wid], idx_v)
    for j in range(NCH):
        p = (wid * NCH + j) * CHUNK
        pltpu.async_copy(h_ent_hbm.at[idx_v.at[j]], rows_v, sem).wait()
        pltpu.sync_copy(rows_v, xs_hbm.at[pl.ds(p, CHUNK)])


def _sc_scatter_rows(tgt_idx_hbm, ys_hbm, ynat_hbm, idx_v, rows_v, sem):
    wid = _wid()
    pltpu.sync_copy(tgt_idx_hbm.at[wid], idx_v)
    for j in range(NCH):
        p = (wid * NCH + j) * CHUNK
        pltpu.sync_copy(ys_hbm.at[pl.ds(p, CHUNK)], rows_v)
        pltpu.async_copy(rows_v, ynat_hbm.at[idx_v.at[j]], sem).wait()


def _tc_stream_mm(fs_ref, loc_ref, end_ref, wl_ref, sl_ref, ftr_ref, ow_ref,
                  os_ref, fst_ref, ssl_ref, poff_ref, pok_ref,
                  xs_ref, d_ref, w_hbm, ys_ref, wbufs, sems):
    t = pl.program_id(0)

    @pl.when(t == 0)
    def _():
        for j in range(NWBUF):
            @pl.when(pok_ref[j] == 1)
            def _(j=j):
                pltpu.make_async_copy(w_hbm.at[pl.ds(poff_ref[j], WCH)],
                                      wbufs.at[j], sems.at[j]).start()

    s0 = fs_ref[t]
    n = fs_ref[t + 1] - s0
    x = xs_ref[pl.ds(t * 128, 128), :] * d_ref[pl.ds(t * 128, 128), :]
    rid = lax.broadcasted_iota(jnp.int32, (128, 1), 0)

    def body(m, acc):
        q = s0 + m

        @pl.when(ftr_ref[q] == 1)
        def _():
            pltpu.make_async_copy(w_hbm.at[pl.ds(ow_ref[q], WCH)],
                                  wbufs.at[sl_ref[q]],
                                  sems.at[sl_ref[q]]).wait()

            @pl.when(fst_ref[q] == 1)
            def _():
                pltpu.make_async_copy(w_hbm.at[pl.ds(os_ref[q], WCH)],
                                      wbufs.at[ssl_ref[q]],
                                      sems.at[ssl_ref[q]]).start()

        a = loc_ref[q]
        b = end_ref[q]
        xm = jnp.where((rid >= a) & (rid < b), x, 0.0)
        acc = acc + jnp.dot(xm, wbufs[sl_ref[q], wl_ref[q]],
                            preferred_element_type=jnp.float32)
        return acc

    acc = lax.fori_loop(0, n, body, jnp.zeros((128, DIM), jnp.float32))
    ys_ref[pl.ds(t * 128, 128), :] = acc


def _tc_reduce_relu(yn_ref, out_ref):
    acc = yn_ref[0]
    for k in range(1, 6):
        acc = acc + yn_ref[k]
    out_ref[...] = jnp.maximum(acc, 0.0)


def _tc_rel_gcn(at_ref, ha_ref, hb_ref, wr_ref, out_ref):
    hk = [ha_ref[k] + hb_ref[k] for k in range(6)]
    for j in range(6):
        sup = jnp.zeros((NRS, DIM), jnp.float32)
        for k in range(6):
            ajk = at_ref[j, k, :]
            sup = sup + ajk[:, None] * hk[k]
        out_ref[j] = jnp.maximum(
            jnp.dot(sup, wr_ref[...], preferred_element_type=jnp.float32), 0.0)


def kernel(epoch, pos_h, pos_r, pos_t, neg_h, neg_r, neg_t, ph_R, ph_D, ph_nn,
           pr_A, pt_R, pt_D, pt_nn, nh_R, nh_D, nh_nn, nr_A, nt_R, nt_D, nt_nn,
           entity_emb, relation_emb, entity_context_table,
           relation_context_table, entity_gcn_weight, relation_gcn_weight,
           entity_adj_table, relation_adj_table):
    f32 = jnp.float32
    i32 = jnp.int32

    all_e = jnp.concatenate([pos_h, pos_t, neg_h, neg_t]).astype(i32)
    all_r = jnp.concatenate([pos_r, neg_r]).astype(i32)
    adjc2 = entity_adj_table[all_e].astype(i32).T.reshape(
        C, NW, EPW).transpose(1, 0, 2)
    radjc2 = relation_adj_table[all_r].astype(i32).T.reshape(
        2 * C, 16, 16).transpose(1, 0, 2)

    rel_flat = jnp.clip(
        jnp.concatenate([ph_R, pt_R, nh_R, nt_R]).reshape(-1).astype(i32),
        0, 2 * R_TOTAL)
    d_flat = jnp.concatenate([ph_D, pt_D, nh_D, nt_D]).reshape(-1)
    order = jnp.argsort(rel_flat).astype(i32)
    sorted_rel = rel_flat[order]
    d_sorted = d_flat[order].reshape(NF, 1)
    src_sorted = ((order // 36) * 6 + order % 6).astype(i32).reshape(
        NW, NCH, CHUNK)
    tgt_sorted = ((order % 6) * (NES * 6) + (order // 36) * 6
                  + (order % 36) // 6).astype(i32).reshape(NW, NCH, CHUNK)

    ii = jnp.arange(NF, dtype=i32)
    change = jnp.concatenate(
        [jnp.ones((1,), bool), sorted_rel[1:] != sorted_rel[:-1]])
    flag = change | (ii % 128 == 0)
    starts = jnp.nonzero(flag, size=NSEG, fill_value=NF)[0].astype(i32)
    seg_rel = jnp.where(starts < NF,
                        sorted_rel[jnp.clip(starts, 0, NF - 1)],
                        2 * R_TOTAL).astype(i32)
    seg_loc = (starts % 128).astype(i32)
    ends = jnp.concatenate([starts[1:], jnp.array([NF], i32)])
    seg_end = seg_loc + (ends - starts)
    first_seg = jnp.searchsorted(
        starts, jnp.arange(NF // 128 + 1, dtype=i32) * 128).astype(i32)
    cs = seg_rel // WCH
    off = jnp.minimum(cs * WCH, 2 * R_TOTAL + 1 - WCH).astype(i32)
    swloc = (seg_rel - off).astype(i32)
    ftrans = jnp.concatenate(
        [jnp.ones((1,), i32), (cs[1:] != cs[:-1]).astype(i32)])
    k_of = jnp.cumsum(ftrans).astype(i32) - 1
    kmax = k_of[-1]
    sslot = (k_of % NWBUF).astype(i32)
    tr_idx = jnp.nonzero(ftrans, size=NTR, fill_value=NSEG - 1)[0]
    seq_off = off[tr_idx]
    sostart = seq_off[jnp.clip(k_of + NWBUF - 1, 0, NTR - 1)].astype(i32)
    sfstart = (ftrans.astype(bool) & (k_of + NWBUF - 1 <= kmax)
               & (k_of >= 1)).astype(i32)
    sstart_slot = ((k_of + NWBUF - 1) % NWBUF).astype(i32)
    prime_off = seq_off[:NWBUF].astype(i32)
    prime_ok = (jnp.arange(NWBUF) <= kmax).astype(i32)

    mesh = plsc.VectorSubcoreMesh(core_axis_name="c", subcore_axis_name="s")

    h_ent, h_rel_a, h_rel_b = pl.kernel(
        _sc_build_h,
        out_type=[jax.ShapeDtypeStruct((NES * 6, DIM), f32),
                  jax.ShapeDtypeStruct((6 * NRS, DIM), f32),
                  jax.ShapeDtypeStruct((6 * NRS, DIM), f32)],
        mesh=mesh,
        scratch_types=[pltpu.VMEM((16,), i32),
                       pltpu.VMEM((C, 16), i32),
                       pltpu.VMEM((2 * C, 16), i32),
                       pltpu.VMEM((12, 16, DIM), f32),
                       pltpu.SemaphoreType.DMA],
    )(all_e, adjc2, entity_emb, entity_context_table,
      all_r, radjc2, relation_emb, relation_context_table)

    xs = pl.kernel(
        _sc_gather_rows,
        out_type=jax.ShapeDtypeStruct((NF, DIM), f32),
        mesh=mesh,
        scratch_types=[pltpu.VMEM((NCH, CHUNK), i32),
                       pltpu.VMEM((CHUNK, DIM), f32),
                       pltpu.SemaphoreType.DMA],
    )(src_sorted, h_ent)

    grid_spec = pltpu.PrefetchScalarGridSpec(
        num_scalar_prefetch=12,
        grid=(NF // 128,),
        in_specs=[
            pl.BlockSpec((NF, DIM), lambda i, *_: (0, 0)),
            pl.BlockSpec((NF, 1), lambda i, *_: (0, 0)),
            pl.BlockSpec(memory_space=pl.ANY),
        ],
        out_specs=pl.BlockSpec((NF, DIM), lambda i, *_: (0, 0)),
        scratch_shapes=[pltpu.VMEM((NWBUF, WCH, DIM, DIM), f32),
                        pltpu.SemaphoreType.DMA((NWBUF,))],
    )
    ys = pl.pallas_call(
        _tc_stream_mm,
        grid_spec=grid_spec,
        out_shape=jax.ShapeDtypeStruct((NF, DIM), f32),
        compiler_params=pltpu.CompilerParams(
            dimension_semantics=("arbitrary",)),
    )(first_seg, seg_loc, seg_end, swloc, sslot, ftrans, off,
      sostart, sfstart, sstart_slot, prime_off, prime_ok,
      xs, d_sorted, entity_gcn_weight)

    ynat = pl.kernel(
        _sc_scatter_rows,
        out_type=jax.ShapeDtypeStruct((NF, DIM), f32),
        mesh=mesh,
        scratch_types=[pltpu.VMEM((NCH, CHUNK), i32),
                       pltpu.VMEM((CHUNK, DIM), f32),
                       pltpu.SemaphoreType.DMA],
    )(tgt_sorted, ys)

    yn3 = ynat.reshape(6, NES * 6, DIM)
    ent_out = pl.pallas_call(
        _tc_reduce_relu,
        grid=(8,),
        in_specs=[pl.BlockSpec((6, NES * 6 // 8, DIM), lambda i: (0, i, 0))],
        out_specs=pl.BlockSpec((NES * 6 // 8, DIM), lambda i: (i, 0)),
        out_shape=jax.ShapeDtypeStruct((NES * 6, DIM), f32),
    )(yn3)
    eo = ent_out.reshape(4, B, 6, DIM)

    a_t = jnp.concatenate([pr_A, nr_A]).transpose(1, 2, 0)
    rel_out = pl.pallas_call(
        _tc_rel_gcn,
        out_shape=jax.ShapeDtypeStruct((6, NRS, DIM), f32),
    )(a_t, h_rel_a.reshape(6, NRS, DIM), h_rel_b.reshape(6, NRS, DIM),
      relation_gcn_weight)
    ro = rel_out.transpose(1, 0, 2)

    return (eo[0], eo[1], eo[2], eo[3], ro[:B], ro[B:])


# speedup vs baseline: 1.0782x; 1.0111x over previous
import jax
import jax.numpy as jnp
from jax import lax
from jax.experimental import pallas as pl
from jax.experimental.pallas import tpu as pltpu
from jax.experimental.pallas import tpu_sc as plsc

E_TOTAL = 100000
R_TOTAL = 500
DIM = 128
C = 5
B = 128
NES = 4 * B
NRS = 2 * B
NF = NES * 36
NSEG = 1160
WCH = 32
NWBUF = 4
NTR = 36
NW = 32
EPW = NES // NW
CHUNK = 96
NCH = NF // (NW * CHUNK)


def _wid():
    return lax.axis_index("s") * 2 + lax.axis_index("c")


def _sc_build_h(all_e_hbm, adjc2_hbm, emb_hbm, ect_hbm,
                all_r_hbm, radjc2_hbm, remb_hbm, rct_hbm,
                h_ent_hbm, h_rel_a_hbm, h_rel_b_hbm,
                idx_v, idx5_v, idx10_v, buf, sem):
    wid = _wid()
    iota = lax.iota(jnp.int32, 16)
    base = wid * EPW
    pltpu.sync_copy(all_e_hbm.at[pl.ds(base, 16)], idx_v)
    pltpu.sync_copy(adjc2_hbm.at[wid], idx5_v)
    tgt0 = (iota + base) * 6

    def g_ent(c):
        if c == 0:
            return pltpu.make_async_copy(emb_hbm.at[idx_v], buf.at[0], sem)
        return pltpu.make_async_copy(ect_hbm.at[idx5_v.at[c - 1]], buf.at[c],
                                     sem)

    def s_ent(c):
        return pltpu.make_async_copy(buf.at[c], h_ent_hbm.at[tgt0 + c], sem)

    for c in range(6):
        g_ent(c).start()
    for c in range(6):
        g_ent(c).wait()
    for c in range(6):
        s_ent(c).start()
    for c in range(6):
        s_ent(c).wait()

    @pl.when(wid < 16)
    def _():
        rbase = wid * 16
        pltpu.sync_copy(all_r_hbm.at[pl.ds(rbase, 16)], idx_v)
        pltpu.sync_copy(radjc2_hbm.at[wid], idx10_v)
        rtgt = rbase + iota
        zcol = jnp.full((16,), R_TOTAL, jnp.int32)

        def g_rel(j):
            if j == 0:
                return pltpu.make_async_copy(remb_hbm.at[idx_v], buf.at[0],
                                             sem)
            if j == 11:
                return pltpu.make_async_copy(rct_hbm.at[zcol], buf.at[11],
                                             sem)
            return pltpu.make_async_copy(rct_hbm.at[idx10_v.at[j - 1]],
                                         buf.at[j], sem)

        def s_rel(j):
            if j == 0:
                dst = h_rel_a_hbm.at[rtgt]
            elif j == 11:
                dst = h_rel_b_hbm.at[rtgt]
            elif j % 2 == 1:
                dst = h_rel_a_hbm.at[(1 + (j - 1) // 2) * NRS + rtgt]
            else:
                dst = h_rel_b_hbm.at[(1 + (j - 2) // 2) * NRS + rtgt]
            return pltpu.make_async_copy(buf.at[j], dst, sem)

        for j in range(12):
            g_rel(j).start()
        for j in range(12):
            g_rel(j).wait()
        for j in range(12):
            s_rel(j).start()
        for j in range(12):
            s_rel(j).wait()


def _sc_gather_rows(src_idx_hbm, h_ent_hbm, xs_hbm, idx_v, rows_v, sem):
    wid = _wid()
    pltpu.sync_copy(src_idx_hbm.at[wid], idx_v)

    def gth(j):
        return pltpu.make_async_copy(h_ent_hbm.at[idx_v.at[j]], rows_v.at[j],
                                     sem)

    def put(j):
        p = (wid * NCH + j) * CHUNK
        return pltpu.make_async_copy(rows_v.at[j], xs_hbm.at[pl.ds(p, CHUNK)],
                                     sem)

    for j in range(NCH):
        gth(j).start()
    for j in range(NCH):
        gth(j).wait()
    for j in range(NCH):
        put(j).start()
    for j in range(NCH):
        put(j).wait()


def _sc_scatter_rows(tgt_idx_hbm, ys_hbm, ynat_hbm, idx_v, rows_v, sem):
    wid = _wid()
    pltpu.sync_copy(tgt_idx_hbm.at[wid], idx_v)

    def get(j):
        p = (wid * NCH + j) * CHUNK
        return pltpu.make_async_copy(ys_hbm.at[pl.ds(p, CHUNK)], rows_v.at[j],
                                     sem)

    def sct(j):
        return pltpu.make_async_copy(rows_v.at[j], ynat_hbm.at[idx_v.at[j]],
                                     sem)

    for j in range(NCH):
        get(j).start()
    for j in range(NCH):
        get(j).wait()
    for j in range(NCH):
        sct(j).start()
    for j in range(NCH):
        sct(j).wait()


def _tc_stream_mm(fs_ref, loc_ref, end_ref, wl_ref, sl_ref, ftr_ref, ow_ref,
                  os_ref, fst_ref, ssl_ref, poff_ref, pok_ref,
                  xs_ref, d_ref, w_hbm, ys_ref, wbufs, sems):
    t = pl.program_id(0)

    @pl.when(t == 0)
    def _():
        for j in range(NWBUF):
            @pl.when(pok_ref[j] == 1)
            def _(j=j):
                pltpu.make_async_copy(w_hbm.at[pl.ds(poff_ref[j], WCH)],
                                      wbufs.at[j], sems.at[j]).start()

    s0 = fs_ref[t]
    n = fs_ref[t + 1] - s0
    x = xs_ref[pl.ds(t * 128, 128), :] * d_ref[pl.ds(t * 128, 128), :]
    rid = lax.broadcasted_iota(jnp.int32, (128, 1), 0)

    def body(m, acc):
        q = s0 + m

        @pl.when(ftr_ref[q] == 1)
        def _():
            pltpu.make_async_copy(w_hbm.at[pl.ds(ow_ref[q], WCH)],
                                  wbufs.at[sl_ref[q]],
                                  sems.at[sl_ref[q]]).wait()

            @pl.when(fst_ref[q] == 1)
            def _():
                pltpu.make_async_copy(w_hbm.at[pl.ds(os_ref[q], WCH)],
                                      wbufs.at[ssl_ref[q]],
                                      sems.at[ssl_ref[q]]).start()

        a = loc_ref[q]
        b = end_ref[q]
        xm = jnp.where((rid >= a) & (rid < b), x, 0.0)
        acc = acc + jnp.dot(xm, wbufs[sl_ref[q], wl_ref[q]],
                            preferred_element_type=jnp.float32)
        return acc

    acc = lax.fori_loop(0, n, body, jnp.zeros((128, DIM), jnp.float32))
    ys_ref[pl.ds(t * 128, 128), :] = acc


def _tc_reduce_relu(yn_ref, out_ref):
    acc = yn_ref[0]
    for k in range(1, 6):
        acc = acc + yn_ref[k]
    out_ref[...] = jnp.maximum(acc, 0.0)


def _tc_rel_gcn(at_ref, ha_ref, hb_ref, wr_ref, out_ref):
    hk = [ha_ref[k] + hb_ref[k] for k in range(6)]
    for j in range(6):
        sup = jnp.zeros((NRS, DIM), jnp.float32)
        for k in range(6):
            ajk = at_ref[j, k, :]
            sup = sup + ajk[:, None] * hk[k]
        out_ref[j] = jnp.maximum(
            jnp.dot(sup, wr_ref[...], preferred_element_type=jnp.float32), 0.0)


def kernel(epoch, pos_h, pos_r, pos_t, neg_h, neg_r, neg_t, ph_R, ph_D, ph_nn,
           pr_A, pt_R, pt_D, pt_nn, nh_R, nh_D, nh_nn, nr_A, nt_R, nt_D, nt_nn,
           entity_emb, relation_emb, entity_context_table,
           relation_context_table, entity_gcn_weight, relation_gcn_weight,
           entity_adj_table, relation_adj_table):
    f32 = jnp.float32
    i32 = jnp.int32

    all_e = jnp.concatenate([pos_h, pos_t, neg_h, neg_t]).astype(i32)
    all_r = jnp.concatenate([pos_r, neg_r]).astype(i32)
    adjc2 = entity_adj_table[all_e].astype(i32).T.reshape(
        C, NW, EPW).transpose(1, 0, 2)
    radjc2 = relation_adj_table[all_r].astype(i32).T.reshape(
        2 * C, 16, 16).transpose(1, 0, 2)

    rel_flat = jnp.clip(
        jnp.concatenate([ph_R, pt_R, nh_R, nt_R]).reshape(-1).astype(i32),
        0, 2 * R_TOTAL)
    d_flat = jnp.concatenate([ph_D, pt_D, nh_D, nt_D]).reshape(-1)
    order = jnp.argsort(rel_flat).astype(i32)
    sorted_rel = rel_flat[order]
    d_sorted = d_flat[order].reshape(NF, 1)
    src_sorted = ((order // 36) * 6 + order % 6).astype(i32).reshape(
        NW, NCH, CHUNK)
    tgt_sorted = ((order % 6) * (NES * 6) + (order // 36) * 6
                  + (order % 36) // 6).astype(i32).reshape(NW, NCH, CHUNK)

    ii = jnp.arange(NF, dtype=i32)
    change = jnp.concatenate(
        [jnp.ones((1,), bool), sorted_rel[1:] != sorted_rel[:-1]])
    flag = change | (ii % 128 == 0)
    starts = jnp.nonzero(flag, size=NSEG, fill_value=NF)[0].astype(i32)
    seg_rel = jnp.where(starts < NF,
                        sorted_rel[jnp.clip(starts, 0, NF - 1)],
                        2 * R_TOTAL).astype(i32)
    seg_loc = (starts % 128).astype(i32)
    ends = jnp.concatenate([starts[1:], jnp.array([NF], i32)])
    seg_end = seg_loc + (ends - starts)
    first_seg = jnp.searchsorted(
        starts, jnp.arange(NF // 128 + 1, dtype=i32) * 128).astype(i32)
    cs = seg_rel // WCH
    off = jnp.minimum(cs * WCH, 2 * R_TOTAL + 1 - WCH).astype(i32)
    swloc = (seg_rel - off).astype(i32)
    ftrans = jnp.concatenate(
        [jnp.ones((1,), i32), (cs[1:] != cs[:-1]).astype(i32)])
    k_of = jnp.cumsum(ftrans).astype(i32) - 1
    kmax = k_of[-1]
    sslot = (k_of % NWBUF).astype(i32)
    tr_idx = jnp.nonzero(ftrans, size=NTR, fill_value=NSEG - 1)[0]
    seq_off = off[tr_idx]
    sostart = seq_off[jnp.clip(k_of + NWBUF - 1, 0, NTR - 1)].astype(i32)
    sfstart = (ftrans.astype(bool) & (k_of + NWBUF - 1 <= kmax)
               & (k_of >= 1)).astype(i32)
    sstart_slot = ((k_of + NWBUF - 1) % NWBUF).astype(i32)
    prime_off = seq_off[:NWBUF].astype(i32)
    prime_ok = (jnp.arange(NWBUF) <= kmax).astype(i32)

    mesh = plsc.VectorSubcoreMesh(core_axis_name="c", subcore_axis_name="s")

    h_ent, h_rel_a, h_rel_b = pl.kernel(
        _sc_build_h,
        out_type=[jax.ShapeDtypeStruct((NES * 6, DIM), f32),
                  jax.ShapeDtypeStruct((6 * NRS, DIM), f32),
                  jax.ShapeDtypeStruct((6 * NRS, DIM), f32)],
        mesh=mesh,
        scratch_types=[pltpu.VMEM((16,), i32),
                       pltpu.VMEM((C, 16), i32),
                       pltpu.VMEM((2 * C, 16), i32),
                       pltpu.VMEM((12, 16, DIM), f32),
                       pltpu.SemaphoreType.DMA],
    )(all_e, adjc2, entity_emb, entity_context_table,
      all_r, radjc2, relation_emb, relation_context_table)

    xs = pl.kernel(
        _sc_gather_rows,
        out_type=jax.ShapeDtypeStruct((NF, DIM), f32),
        mesh=mesh,
        scratch_types=[pltpu.VMEM((NCH, CHUNK), i32),
                       pltpu.VMEM((NCH, CHUNK, DIM), f32),
                       pltpu.SemaphoreType.DMA],
    )(src_sorted, h_ent)

    grid_spec = pltpu.PrefetchScalarGridSpec(
        num_scalar_prefetch=12,
        grid=(NF // 128,),
        in_specs=[
            pl.BlockSpec((NF, DIM), lambda i, *_: (0, 0)),
            pl.BlockSpec((NF, 1), lambda i, *_: (0, 0)),
            pl.BlockSpec(memory_space=pl.ANY),
        ],
        out_specs=pl.BlockSpec((NF, DIM), lambda i, *_: (0, 0)),
        scratch_shapes=[pltpu.VMEM((NWBUF, WCH, DIM, DIM), f32),
                        pltpu.SemaphoreType.DMA((NWBUF,))],
    )
    ys = pl.pallas_call(
        _tc_stream_mm,
        grid_spec=grid_spec,
        out_shape=jax.ShapeDtypeStruct((NF, DIM), f32),
        compiler_params=pltpu.CompilerParams(
            dimension_semantics=("arbitrary",)),
    )(first_seg, seg_loc, seg_end, swloc, sslot, ftrans, off,
      sostart, sfstart, sstart_slot, prime_off, prime_ok,
      xs, d_sorted, entity_gcn_weight)

    ynat = pl.kernel(
        _sc_scatter_rows,
        out_type=jax.ShapeDtypeStruct((NF, DIM), f32),
        mesh=mesh,
        scratch_types=[pltpu.VMEM((NCH, CHUNK), i32),
                       pltpu.VMEM((NCH, CHUNK, DIM), f32),
                       pltpu.SemaphoreType.DMA],
    )(tgt_sorted, ys)

    yn3 = ynat.reshape(6, NES * 6, DIM)
    ent_out = pl.pallas_call(
        _tc_reduce_relu,
        grid=(8,),
        in_specs=[pl.BlockSpec((6, NES * 6 // 8, DIM), lambda i: (0, i, 0))],
        out_specs=pl.BlockSpec((NES * 6 // 8, DIM), lambda i: (i, 0)),
        out_shape=jax.ShapeDtypeStruct((NES * 6, DIM), f32),
    )(yn3)
    eo = ent_out.reshape(4, B, 6, DIM)

    a_t = jnp.concatenate([pr_A, nr_A]).transpose(1, 2, 0)
    rel_out = pl.pallas_call(
        _tc_rel_gcn,
        out_shape=jax.ShapeDtypeStruct((6, NRS, DIM), f32),
    )(a_t, h_rel_a.reshape(6, NRS, DIM), h_rel_b.reshape(6, NRS, DIM),
      relation_gcn_weight)
    ro = rel_out.transpose(1, 0, 2)

    return (eo[0], eo[1], eo[2], eo[3], ro[:B], ro[B:])
